# baseline (speedup 1.0000x reference)
import math

import jax
import jax.numpy as jnp
from jax import lax
from jax.experimental import pallas as pl

N = 20000
NPAD = 20480
ROWS = 160
LANES = 128
NUM_CLASSES = 21
NCLS = NUM_CLASSES - 1
IMG_H, IMG_W = 800.0, 1216.0
SCORE_THRESH = 0.05
NMS_THRESH = 0.5
NUM_DETECTIONS = 100
MIN_SIZE = 1.0
K_PRE = 1000
KP = 1024
DETP = 128
BBOX_XFORM_CLIP = math.log(1000.0 / 16.0)
INT_MIN = -2147483648
NEG1_KEY = -1082130432
_HI = jax.lax.Precision.HIGHEST


def _dot(a, b, dims):
    return lax.dot_general(a, b, (dims, ((), ())), precision=_HI)


def _score_decode_body(logit_ref, reg_ref, prop_ref, pay_ref):
    lg = logit_ref[...]
    m = jnp.max(lg, axis=0, keepdims=True)
    e = jnp.exp(lg - m)
    z = jnp.sum(e, axis=0, keepdims=True)
    prob = e / z

    pr = prop_ref[...]
    px1, py1 = pr[0:1], pr[1:2]
    px2, py2 = pr[2:3], pr[3:4]
    w = px2 - px1
    h = py2 - py1
    cx = px1 + 0.5 * w
    cy = py1 + 0.5 * h

    for c in range(NCLS):
        d = reg_ref[c + 1]
        dx = d[0:1] / 10.0
        dy = d[1:2] / 10.0
        dw = jnp.minimum(d[2:3] / 5.0, BBOX_XFORM_CLIP)
        dh = jnp.minimum(d[3:4] / 5.0, BBOX_XFORM_CLIP)
        pcx = dx * w + cx
        pcy = dy * h + cy
        pw = jnp.exp(dw) * w
        ph = jnp.exp(dh) * h
        x1 = jnp.clip(pcx - 0.5 * pw, 0.0, IMG_W)
        y1 = jnp.clip(pcy - 0.5 * ph, 0.0, IMG_H)
        x2 = jnp.clip(pcx + 0.5 * pw, 0.0, IMG_W)
        y2 = jnp.clip(pcy + 0.5 * ph, 0.0, IMG_H)
        bw = x2 - x1
        bh = y2 - y1
        sc = prob[c + 1:c + 2]
        valid = (sc >= SCORE_THRESH) & (bw >= MIN_SIZE) & (bh >= MIN_SIZE)
        s = jnp.where(valid, sc, -1.0)
        pay_ref[c] = jnp.concatenate([x1, y1, x2, y2, s], axis=0)


def _select_body(s_ref, pos_ref):
    sv = s_ref[...]
    r_i = lax.broadcasted_iota(jnp.int32, (ROWS, LANES), 0)
    l_i = lax.broadcasted_iota(jnp.int32, (ROWS, LANES), 1)
    ii = LANES * r_i + l_i
    key = jnp.where((ii < N)[None], lax.bitcast_convert_type(sv, jnp.int32),
                    INT_MIN)

    cnt0 = jnp.sum((key >= 0).astype(jnp.int32), axis=(1, 2), keepdims=True)

    def step(i, p):
        bitv = lax.shift_left(jnp.int32(1), jnp.int32(30) - i)
        cand = p | bitv
        cnt = jnp.sum((key >= cand).astype(jnp.int32), axis=(1, 2),
                      keepdims=True)
        return jnp.where(cnt >= K_PRE, cand, p)

    p0 = jnp.zeros((NCLS, 1, 1), jnp.int32)
    p = lax.fori_loop(0, 31, step, p0)
    v_star = jnp.where(cnt0 >= K_PRE, p, NEG1_KEY)
    g = jnp.sum((key > v_star).astype(jnp.int32), axis=(1, 2), keepdims=True)
    t = (K_PRE - g).astype(jnp.float32)

    u_incl = (lax.broadcasted_iota(jnp.int32, (LANES, LANES), 0)
              <= lax.broadcasted_iota(jnp.int32, (LANES, LANES), 1)
              ).astype(jnp.float32)
    l_strict = (lax.broadcasted_iota(jnp.int32, (ROWS, ROWS), 0)
                > lax.broadcasted_iota(jnp.int32, (ROWS, ROWS), 1)
                ).astype(jnp.float32)

    def prefix_incl(mask_f):
        intra = _dot(mask_f, u_incl, ((1,), (0,)))
        rowt = intra[:, LANES - 1:LANES]
        offs = _dot(l_strict, rowt, ((1,), (0,)))
        return intra + offs

    for c in range(NCLS):
        kc = key[c]
        vsc = v_star[c]
        tie = kc == vsc
        trank = prefix_incl(tie.astype(jnp.float32))
        sel = (kc > vsc) | (tie & (trank <= t[c]))
        posi = prefix_incl(sel.astype(jnp.float32)).astype(jnp.int32) - 1
        pos_ref[c] = jnp.where(sel, posi, 100000)


def _nms_body(pay_ref, pos_ref, dbox_ref, ds_ref):
    pay = pay_ref[0]
    posr = pos_ref[0]

    lane_col = lax.broadcasted_iota(jnp.int32, (LANES, 4096), 0)[:, 0:1]
    accs = [jnp.zeros((5, LANES), jnp.float32) for _ in range(8)]
    for q in range(NPAD // 4096):
        sl = slice(q * 4096, (q + 1) * 4096)
        pq = posr[:, sl]
        payq = pay[:, sl]
        ohl = (((pq & 127) == lane_col)).astype(jnp.float32)
        adiv = pq >> 7
        for a in range(8):
            mask = (adiv == a).astype(jnp.float32)
            accs[a] = accs[a] + _dot(payq * mask, ohl, ((1,), (1,)))
    rowp = jnp.concatenate(accs, axis=1)

    im = lax.broadcasted_iota(jnp.int32, (KP, KP), 0)
    jm = lax.broadcasted_iota(jnp.int32, (KP, KP), 1)
    upper = im < jm
    eye = (im == jm).astype(jnp.float32)
    ones_row = jnp.ones((1, KP), jnp.float32)

    def to_col(row):
        return _dot(eye, row, ((1,), (1,)))

    s_row = rowp[4:5]
    s_col = to_col(s_row)
    amat = jnp.where((s_col > s_row) | ((s_col == s_row) & upper), 1.0, 0.0)
    rank = _dot(ones_row, amat, ((1,), (0,)))
    p1t = (to_col(rank).astype(jnp.int32) == lax.broadcasted_iota(
        jnp.int32, (1, KP), 1)).astype(jnp.float32)
    srt = _dot(rowp, p1t, ((1,), (0,)))

    bcol = to_col(srt[0:4])
    x1c, y1c = bcol[:, 0:1], bcol[:, 1:2]
    x2c, y2c = bcol[:, 2:3], bcol[:, 3:4]
    x1r, y1r = srt[0:1], srt[1:2]
    x2r, y2r = srt[2:3], srt[3:4]
    s_srt = srt[4:5]

    area_c = (x2c - x1c) * (y2c - y1c)
    area_r = (x2r - x1r) * (y2r - y1r)
    wx = jnp.maximum(jnp.minimum(x2c, x2r) - jnp.maximum(x1c, x1r), 0.0)
    wy = jnp.maximum(jnp.minimum(y2c, y2r) - jnp.maximum(y1c, y1r), 0.0)
    inter = wx * wy
    union = area_c + area_r - inter
    iou = inter / jnp.maximum(union, 1e-9)
    sup_f = jnp.where((iou > NMS_THRESH) & upper, 1.0, 0.0)

    def cond(c):
        return c[1]

    def body(c):
        k = c[0]
        sup = _dot(k, sup_f, ((1,), (0,)))
        kn = jnp.where(sup > 0.5, 0.0, 1.0)
        return kn, jnp.any(kn != k)

    keep0 = jnp.ones((1, KP), jnp.float32)
    keep, _ = lax.while_loop(cond, body, (keep0, True))

    kv = (keep > 0.5) & (s_srt > 0.0)
    s2 = jnp.where(kv, s_srt, -1.0)
    s2_col = to_col(s2)
    amat2 = jnp.where((s2_col > s2) | ((s2_col == s2) & upper), 1.0, 0.0)
    rank2 = _dot(ones_row, amat2, ((1,), (0,)))
    ri = rank2.astype(jnp.int32)
    p_col = lax.broadcasted_iota(jnp.int32, (DETP, KP), 0)
    perm = jnp.where(ri == p_col, 1.0, 0.0)

    det_box = _dot(perm, bcol, ((1,), (0,)))
    det_s = _dot(perm, s2_col, ((1,), (0,)))
    dvalid = det_s > 0.0
    dbox_ref[0] = jnp.where(dvalid, det_box, 0.0)
    ds_ref[0] = jnp.where(dvalid, det_s, 0.0)


@jax.jit
def kernel(class_logit, box_regression, proposal):
    logit_t = jnp.pad(class_logit.T, ((0, 0), (0, NPAD - N)))
    reg_t = jnp.pad(
        jnp.transpose(box_regression.reshape(N, NUM_CLASSES, 4), (1, 2, 0)),
        ((0, 0), (0, 0), (0, NPAD - N)))
    prop_t = jnp.pad(proposal.T, ((0, 0), (0, NPAD - N)))

    blk = 2048
    payload = pl.pallas_call(
        _score_decode_body,
        grid=(NPAD // blk,),
        in_specs=[
            pl.BlockSpec((NUM_CLASSES, blk), lambda i: (0, i)),
            pl.BlockSpec((NUM_CLASSES, 4, blk), lambda i: (0, 0, i)),
            pl.BlockSpec((4, blk), lambda i: (0, i)),
        ],
        out_specs=pl.BlockSpec((NCLS, 5, blk), lambda i: (0, 0, i)),
        out_shape=jax.ShapeDtypeStruct((NCLS, 5, NPAD), jnp.float32),
    )(logit_t, reg_t, prop_t)

    pos = pl.pallas_call(
        _select_body,
        grid=(1,),
        in_specs=[pl.BlockSpec((NCLS, ROWS, LANES), lambda i: (0, 0, 0))],
        out_specs=pl.BlockSpec((NCLS, ROWS, LANES), lambda i: (0, 0, 0)),
        out_shape=jax.ShapeDtypeStruct((NCLS, ROWS, LANES), jnp.int32),
    )(payload[:, 4, :].reshape(NCLS, ROWS, LANES))

    det_box, det_s = pl.pallas_call(
        _nms_body,
        grid=(NCLS,),
        in_specs=[
            pl.BlockSpec((1, 5, NPAD), lambda c: (c, 0, 0)),
            pl.BlockSpec((1, 1, NPAD), lambda c: (c, 0, 0)),
        ],
        out_specs=[
            pl.BlockSpec((1, DETP, 4), lambda c: (c, 0, 0)),
            pl.BlockSpec((1, DETP, 1), lambda c: (c, 0, 0)),
        ],
        out_shape=[
            jax.ShapeDtypeStruct((NCLS, DETP, 4), jnp.float32),
            jax.ShapeDtypeStruct((NCLS, DETP, 1), jnp.float32),
        ],
    )(payload, pos.reshape(NCLS, 1, NPAD))

    db = det_box[:, :NUM_DETECTIONS, :]
    ds = det_s[:, :NUM_DETECTIONS, 0]
    labels = jnp.broadcast_to(
        jnp.arange(1, NUM_CLASSES, dtype=jnp.float32)[:, None],
        (NCLS, NUM_DETECTIONS))
    labels = jnp.where(ds > 0.0, labels, 0.0)
    det = jnp.concatenate(
        [db.reshape(-1, 4), ds.reshape(-1, 1), labels.reshape(-1, 1)], axis=1)
    return det
